# baseline (speedup 1.0000x reference)
import functools

import jax
import jax.numpy as jnp
from jax import lax
from jax.experimental import pallas as pl
from jax.experimental.pallas import tpu as pltpu
from jax.experimental.pallas import tpu_sc as plsc

GAMMA = 0.1
EPS = 0.1
NUM_LAYERS = 4

_NC = 2
_NS = 16
_NW = _NC * _NS
_CHUNK = 128


def _dotT(a, b):
  return lax.dot_general(a, b, (((1,), (1,)), ((), ())),
                         preferred_element_type=jnp.float32)


def _dot(a, b):
  return lax.dot_general(a, b, (((1,), (0,)), ((), ())),
                         preferred_element_type=jnp.float32)


_GC = 256
_ZR = 64


@functools.lru_cache(maxsize=None)
def _make_agg(NP, EP0, EP1, H):
  ept0 = EP0 // _NS
  ept1 = EP1 // _NS
  rpt = NP // _NS
  mesh = plsc.VectorSubcoreMesh(core_axis_name="c", subcore_axis_name="s",
                                num_cores=_NC, num_subcores=_NS)

  @functools.partial(
      pl.kernel,
      out_type=jax.ShapeDtypeStruct((_NC * NP, H), jnp.float32),
      mesh=mesh,
      scratch_types=[
          pltpu.VMEM((_GC,), jnp.int32),
          pltpu.VMEM((_GC // _CHUNK, _CHUNK), jnp.int32),
          pltpu.VMEM((_GC, H), jnp.float32),
          pltpu.VMEM((_ZR, H), jnp.float32),
          pltpu.VMEM_SHARED((NP, H), jnp.float32),
          pltpu.SemaphoreType.DMA,
      ],
  )
  def agg(hdup_hbm, src_hbm, dst2_hbm, out_hbm, src_v, dst_v, rows_v, zero_v,
          acc_sh, sem):
    c = lax.axis_index("c")
    s = lax.axis_index("s")
    estart = jnp.where(c == 0, s * ept0, EP0 + s * ept1)
    ng_c = jnp.where(c == 0, ept0, ept1) // _GC

    def zr(r, carry):
      for j in range(H // 16):
        zero_v[r, pl.ds(j * 16, 16)] = jnp.zeros((16,), jnp.float32)
      return carry
    lax.fori_loop(0, _ZR, zr, 0)
    for j in range(rpt // _ZR):
      pltpu.sync_copy(zero_v, acc_sh.at[pl.ds(s * rpt + j * _ZR, _ZR)])
    plsc.subcore_barrier()

    def chunk(i, carry):
      base = pl.multiple_of(estart + i * _GC, _GC)
      pltpu.sync_copy(src_hbm.at[pl.ds(base, _GC)], src_v)
      pltpu.sync_copy(
          dst2_hbm.at[pl.ds(pl.multiple_of(base // _CHUNK, _GC // _CHUNK),
                            _GC // _CHUNK)], dst_v)
      pltpu.async_copy(hdup_hbm.at[src_v], rows_v, sem).wait()
      for j in range(_GC // _CHUNK):
        pltpu.sync_copy(rows_v.at[pl.ds(j * _CHUNK, _CHUNK)],
                        acc_sh.at[dst_v.at[j]], add=True)
      return carry
    lax.fori_loop(0, ng_c, chunk, 0)

    plsc.subcore_barrier()
    pltpu.sync_copy(acc_sh.at[pl.ds(s * rpt, rpt)],
                    out_hbm.at[pl.ds(c * NP + s * rpt, rpt)])

  return agg


def _emb_body(x_ref, w_ref, b_ref, o_ref, o2_ref):
  h = _dotT(x_ref[...], w_ref[...]) + b_ref[...]
  o_ref[...] = h
  o2_ref[...] = jnp.concatenate([h, h], axis=0)


def _layer_body(h_ref, agg_ref, wa_ref, wlin_ref, b_ref, o_ref, o2_ref):
  h = h_ref[...]
  np_ = h.shape[0]
  a = agg_ref[...]
  aggf = a[:np_] + a[np_:]
  hA = _dotT(h, wa_ref[...]) - _dot(h, wa_ref[...]) - GAMMA * h
  neigh = _dotT(aggf, wlin_ref[...])
  conv = hA + neigh + b_ref[...]
  hn = h + EPS * jnp.tanh(conv)
  o_ref[...] = hn
  o2_ref[...] = jnp.concatenate([hn, hn], axis=0)


def _readout_body(h_ref, w1_ref, b1_ref, w2_ref, b2_ref, o_ref):
  t = _dotT(h_ref[...], w1_ref[...]) + b1_ref[...]
  t = jnp.where(t > 0, t, 0.01 * t)
  t = _dotT(t, w2_ref[...]) + b2_ref[...]
  o_ref[...] = jnp.where(t > 0, t, 0.01 * t)


def kernel(x, edge_index, batch, W_emb, b_emb, W_A, bias_conv, W_lin,
           W_r1, b_r1, W_r2, b_r2):
  N, D = x.shape
  H = W_emb.shape[0]
  OUT = W_r2.shape[0]
  E = edge_index.shape[1]

  NP = (N // (_NS * _CHUNK) + 1) * (_NS * _CHUNK)
  grain = _NS * _GC
  EP = -(-E // (_NC * grain)) * (_NC * grain)
  EP0 = EP // 2
  EP1 = EP - EP0

  src = edge_index[0]
  dst = edge_index[1]
  if EP > E:
    pad_dst = N + jnp.arange(EP - E, dtype=jnp.int32) % (NP - N)
    src = jnp.concatenate([src, jnp.zeros((EP - E,), jnp.int32)])
    dst = jnp.concatenate([dst, pad_dst])
  src = jnp.where(jnp.arange(EP) < EP0, src, src + NP)
  dst2 = dst.reshape(EP // _CHUNK, _CHUNK)
  x_p = jnp.pad(x, ((0, NP - N), (0, 0))) if NP > N else x

  agg_call = _make_agg(NP, EP0, EP1, H)

  emb = pl.pallas_call(
      _emb_body,
      out_shape=[jax.ShapeDtypeStruct((NP, H), jnp.float32),
                 jax.ShapeDtypeStruct((2 * NP, H), jnp.float32)])
  layer = pl.pallas_call(
      _layer_body,
      out_shape=[jax.ShapeDtypeStruct((NP, H), jnp.float32),
                 jax.ShapeDtypeStruct((2 * NP, H), jnp.float32)])
  readout = pl.pallas_call(
      _readout_body, out_shape=jax.ShapeDtypeStruct((NP, OUT), jnp.float32))

  h, hdup = emb(x_p, W_emb, b_emb.reshape(1, H))
  for _ in range(NUM_LAYERS):
    parts = agg_call(hdup, src, dst2)
    h, hdup = layer(h, parts, W_A, W_lin, bias_conv.reshape(1, H))
  out = readout(h, W_r1, b_r1.reshape(1, -1), W_r2, b_r2.reshape(1, OUT))
  return out[:N]


# speedup vs baseline: 1.5889x; 1.5889x over previous
import functools

import jax
import jax.numpy as jnp
from jax import lax
from jax.experimental import pallas as pl
from jax.experimental.pallas import tpu as pltpu
from jax.experimental.pallas import tpu_sc as plsc

GAMMA = 0.1
EPS = 0.1
NUM_LAYERS = 4

_NC = 2
_NS = 16
_NW = _NC * _NS
_CHUNK = 128


def _dotT(a, b):
  return lax.dot_general(a, b, (((1,), (1,)), ((), ())),
                         preferred_element_type=jnp.float32)


def _dot(a, b):
  return lax.dot_general(a, b, (((1,), (0,)), ((), ())),
                         preferred_element_type=jnp.float32)


_GC = 128
_ZR = 64


@functools.lru_cache(maxsize=None)
def _make_agg(NP, EP0, EP1, H):
  ept0 = EP0 // _NS
  ept1 = EP1 // _NS
  rpt = NP // _NS
  mesh = plsc.VectorSubcoreMesh(core_axis_name="c", subcore_axis_name="s",
                                num_cores=_NC, num_subcores=_NS)

  @functools.partial(
      pl.kernel,
      out_type=jax.ShapeDtypeStruct((_NC * NP, H), jnp.float32),
      mesh=mesh,
      scratch_types=[
          pltpu.VMEM((_GC,), jnp.int32),
          pltpu.VMEM((_GC // _CHUNK, _CHUNK), jnp.int32),
          pltpu.VMEM((_GC, H), jnp.float32),
          pltpu.VMEM((_ZR, H), jnp.float32),
          pltpu.VMEM_SHARED((NP, H), jnp.float32),
          pltpu.SemaphoreType.DMA,
      ],
  )
  def agg(hdup_hbm, src_hbm, dst2_hbm, out_hbm, src_v, dst_v, rows_v, zero_v,
          acc_sh, sem):
    c = lax.axis_index("c")
    s = lax.axis_index("s")
    estart = jnp.where(c == 0, s * ept0, EP0 + s * ept1)
    ng_c = jnp.where(c == 0, ept0, ept1) // _GC

    def zr(r, carry):
      for j in range(H // 16):
        zero_v[r, pl.ds(j * 16, 16)] = jnp.zeros((16,), jnp.float32)
      return carry
    lax.fori_loop(0, _ZR, zr, 0)
    for j in range(rpt // _ZR):
      pltpu.sync_copy(zero_v, acc_sh.at[pl.ds(s * rpt + j * _ZR, _ZR)])
    plsc.subcore_barrier()

    def chunk(i, carry):
      base = pl.multiple_of(estart + i * _GC, _GC)
      pltpu.sync_copy(src_hbm.at[pl.ds(base, _GC)], src_v)
      pltpu.sync_copy(
          dst2_hbm.at[pl.ds(pl.multiple_of(base // _CHUNK, _GC // _CHUNK),
                            _GC // _CHUNK)], dst_v)
      pltpu.async_copy(hdup_hbm.at[src_v], rows_v, sem).wait()
      for j in range(_GC // _CHUNK):
        pltpu.sync_copy(rows_v.at[pl.ds(j * _CHUNK, _CHUNK)],
                        acc_sh.at[dst_v.at[j]], add=True)
      return carry
    lax.fori_loop(0, ng_c, chunk, 0)

    plsc.subcore_barrier()
    pltpu.sync_copy(acc_sh.at[pl.ds(s * rpt, rpt)],
                    out_hbm.at[pl.ds(c * NP + s * rpt, rpt)])

  return agg


def _emb_body(x_ref, w_ref, b_ref, o_ref, o2_ref):
  h = _dotT(x_ref[...], w_ref[...]) + b_ref[...]
  o_ref[...] = h
  o2_ref[...] = jnp.concatenate([h, h], axis=0)


def _layer_body(h_ref, agg_ref, wa_ref, wlin_ref, b_ref, o_ref, o2_ref):
  h = h_ref[...]
  np_ = h.shape[0]
  a = agg_ref[...]
  aggf = a[:np_] + a[np_:]
  hA = _dotT(h, wa_ref[...]) - _dot(h, wa_ref[...]) - GAMMA * h
  neigh = _dotT(aggf, wlin_ref[...])
  conv = hA + neigh + b_ref[...]
  hn = h + EPS * jnp.tanh(conv)
  o_ref[...] = hn
  o2_ref[...] = jnp.concatenate([hn, hn], axis=0)


def _readout_body(h_ref, w1_ref, b1_ref, w2_ref, b2_ref, o_ref):
  t = _dotT(h_ref[...], w1_ref[...]) + b1_ref[...]
  t = jnp.where(t > 0, t, 0.01 * t)
  t = _dotT(t, w2_ref[...]) + b2_ref[...]
  o_ref[...] = jnp.where(t > 0, t, 0.01 * t)


def kernel(x, edge_index, batch, W_emb, b_emb, W_A, bias_conv, W_lin,
           W_r1, b_r1, W_r2, b_r2):
  N, D = x.shape
  H = W_emb.shape[0]
  OUT = W_r2.shape[0]
  E = edge_index.shape[1]

  NP = (N // (_NS * _CHUNK) + 1) * (_NS * _CHUNK)
  grain = _NS * _GC
  EP = -(-E // (_NC * grain)) * (_NC * grain)
  EP0 = min(max(grain, round(0.6 * EP / grain) * grain), EP - grain)
  EP1 = EP - EP0

  src = edge_index[0]
  dst = edge_index[1]
  if EP > E:
    pad_dst = N + jnp.arange(EP - E, dtype=jnp.int32) % (NP - N)
    src = jnp.concatenate([src, jnp.zeros((EP - E,), jnp.int32)])
    dst = jnp.concatenate([dst, pad_dst])
  src = jnp.where(jnp.arange(EP) < EP0, src, src + NP)
  dst2 = dst.reshape(EP // _CHUNK, _CHUNK)
  x_p = jnp.pad(x, ((0, NP - N), (0, 0))) if NP > N else x

  agg_call = _make_agg(NP, EP0, EP1, H)

  emb = pl.pallas_call(
      _emb_body,
      out_shape=[jax.ShapeDtypeStruct((NP, H), jnp.float32),
                 jax.ShapeDtypeStruct((2 * NP, H), jnp.float32)])
  layer = pl.pallas_call(
      _layer_body,
      out_shape=[jax.ShapeDtypeStruct((NP, H), jnp.float32),
                 jax.ShapeDtypeStruct((2 * NP, H), jnp.float32)])
  readout = pl.pallas_call(
      _readout_body, out_shape=jax.ShapeDtypeStruct((NP, OUT), jnp.float32))

  h, hdup = emb(x_p, W_emb, b_emb.reshape(1, H))
  for _ in range(NUM_LAYERS):
    parts = agg_call(hdup, src, dst2)
    h, hdup = layer(h, parts, W_A, W_lin, bias_conv.reshape(1, H))
  out = readout(h, W_r1, b_r1.reshape(1, -1), W_r2, b_r2.reshape(1, OUT))
  return out[:N]
